# baseline (speedup 1.0000x reference)
import functools

import jax
import jax.numpy as jnp
from jax import lax
from jax.experimental import pallas as pl
from jax.experimental.pallas import tpu as pltpu, tpu_sc as plsc

B = 16384
D = 128
VOCABS = [4, 2, 2, 5, 3, 4, 4]
NT = len(VOCABS)
NROWS = sum(VOCABS)
RPAD = 32
OFFS = [0]
for _v in VOCABS[:-1]:
    OFFS.append(OFFS[-1] + _v)
ROW_TABLE = []
for _i, _v in enumerate(VOCABS):
    ROW_TABLE.extend([_i] * _v)

NC = 2
NS = 16
NW = NC * NS
BPW = B // NW
LANES = 16
NCHUNK = BPW // LANES
NGRP = RPAD // LANES
DUNROLL = 4
MUNROLL = 2
WWORDS = 2 * D * RPAD


def _tanh16(x):
    ax = jnp.abs(x)
    e = jnp.exp(-2.0 * ax)
    return jnp.sign(x) * ((1.0 - e) / (1.0 + e))


def _sc_body(x_hbm, w_hbm, out_hbm, xv, wv, sv, outv, sem):
    wid = lax.axis_index("s") * NC + lax.axis_index("c")
    base = wid * BPW

    copies = [pltpu.async_copy(w_hbm, wv, sem)]
    for i in range(NT):
        copies.append(
            pltpu.async_copy(
                x_hbm.at[pl.ds(i * B + base, BPW)],
                xv.at[pl.ds(i * BPW, BPW)],
                sem,
            )
        )
    for c in copies:
        c.wait()

    def pre_body(it, accs):
        new = list(accs)
        for u in range(DUNROLL):
            off = (it * DUNROLL + u) * RPAD
            for g in range(NGRP):
                evec = wv[pl.ds(off + g * LANES, LANES)]
                pvec = wv[pl.ds(D * RPAD + off + g * LANES, LANES)]
                new[g] = new[g] + _tanh16(evec) * pvec
        return tuple(new)

    zero = jnp.zeros((LANES,), jnp.float32)
    accs = lax.fori_loop(0, D // DUNROLL, pre_body, (zero,) * NGRP)
    for g in range(NGRP):
        sv[pl.ds(g * LANES, LANES)] = accs[g]

    def chunk_body(j, carry):
        for u in range(MUNROLL):
            off = (j * MUNROLL + u) * LANES
            acc = None
            for i in range(NT):
                idx = xv[pl.ds(i * BPW + off, LANES)] + OFFS[i]
                g = plsc.load_gather(sv, [idx])
                acc = g if acc is None else acc + g
            outv[pl.ds(off, LANES)] = acc
        return carry

    lax.fori_loop(0, NCHUNK // MUNROLL, chunk_body, 0)

    pltpu.sync_copy(outv, out_hbm.at[pl.ds(base, BPW)])


@jax.jit
def _run(x, wblob):
    mesh = plsc.VectorSubcoreMesh(core_axis_name="c", subcore_axis_name="s")
    f = functools.partial(
        pl.kernel,
        mesh=mesh,
        out_type=jax.ShapeDtypeStruct((B,), jnp.float32),
        scratch_types=[
            pltpu.VMEM((NT * BPW,), jnp.int32),
            pltpu.VMEM((WWORDS,), jnp.float32),
            pltpu.VMEM((RPAD,), jnp.float32),
            pltpu.VMEM((BPW,), jnp.float32),
            pltpu.SemaphoreType.DMA,
        ],
        compiler_params=pltpu.CompilerParams(needs_layout_passes=False),
    )(_sc_body)
    return f(x, wblob)


def kernel(input, E1, E2, E3, E4, E5, E6, E7, W):
    epk = jnp.concatenate([E1, E2, E3, E4, E5, E6, E7], axis=0)
    wrows = W.reshape(NT, D)[jnp.array(ROW_TABLE)]
    pad = ((0, RPAD - NROWS), (0, 0))
    et = jnp.pad(epk, pad).T.reshape(-1)
    wt = jnp.pad(wrows, pad).T.reshape(-1)
    wblob = jnp.concatenate([et, wt])
    out = _run(input.reshape(-1), wblob)
    return out.reshape(B, 1)


# speedup vs baseline: 1.1176x; 1.1176x over previous
import functools

import jax
import jax.numpy as jnp
from jax import lax
from jax.experimental import pallas as pl
from jax.experimental.pallas import tpu as pltpu, tpu_sc as plsc

B = 16384
D = 128
VOCABS = [4, 2, 2, 5, 3, 4, 4]
NT = len(VOCABS)
NROWS = sum(VOCABS)
RPAD = 32
OFFS = [0]
for _v in VOCABS[:-1]:
    OFFS.append(OFFS[-1] + _v)
ROW_TABLE = []
for _i, _v in enumerate(VOCABS):
    ROW_TABLE.extend([_i] * _v)

NC = 2
NS = 16
NW = NC * NS
BPW = B // NW
LANES = 16
NCHUNK = BPW // LANES
NGRP = RPAD // LANES
DUNROLL = 4
MUNROLL = 2
WWORDS = 2 * D * RPAD


def _tanh16(x):
    ax = jnp.abs(x)
    e = jnp.exp(-2.0 * ax)
    return jnp.sign(x) * ((1.0 - e) / (1.0 + e))


def _sc_body(x_hbm, w_hbm, out_hbm, xv, wv, sv, outv, sem):
    wid = lax.axis_index("s") * NC + lax.axis_index("c")
    base = wid * BPW

    copies = [
        pltpu.async_copy(w_hbm, wv, sem),
        pltpu.async_copy(x_hbm.at[:, pl.ds(base, BPW)], xv, sem),
    ]
    for c in copies:
        c.wait()

    def pre_body(it, accs):
        new = list(accs)
        for u in range(DUNROLL):
            off = (it * DUNROLL + u) * RPAD
            for g in range(NGRP):
                evec = wv[pl.ds(off + g * LANES, LANES)]
                pvec = wv[pl.ds(D * RPAD + off + g * LANES, LANES)]
                new[g] = new[g] + _tanh16(evec) * pvec
        return tuple(new)

    zero = jnp.zeros((LANES,), jnp.float32)
    accs = lax.fori_loop(0, D // DUNROLL, pre_body, (zero,) * NGRP)
    for g in range(NGRP):
        sv[pl.ds(g * LANES, LANES)] = accs[g]

    def chunk_body(j, carry):
        for u in range(MUNROLL):
            off = (j * MUNROLL + u) * LANES
            acc = None
            for i in range(NT):
                idx = xv[i, pl.ds(off, LANES)] + OFFS[i]
                g = plsc.load_gather(sv, [idx])
                acc = g if acc is None else acc + g
            outv[pl.ds(off, LANES)] = acc
        return carry

    lax.fori_loop(0, NCHUNK // MUNROLL, chunk_body, 0)

    pltpu.sync_copy(outv, out_hbm.at[pl.ds(base, BPW)])


@jax.jit
def _run(x, wblob):
    mesh = plsc.VectorSubcoreMesh(core_axis_name="c", subcore_axis_name="s")
    f = functools.partial(
        pl.kernel,
        mesh=mesh,
        out_type=jax.ShapeDtypeStruct((B,), jnp.float32),
        scratch_types=[
            pltpu.VMEM((NT, BPW), jnp.int32),
            pltpu.VMEM((WWORDS,), jnp.float32),
            pltpu.VMEM((RPAD,), jnp.float32),
            pltpu.VMEM((BPW,), jnp.float32),
            pltpu.SemaphoreType.DMA,
        ],
        compiler_params=pltpu.CompilerParams(needs_layout_passes=False),
    )(_sc_body)
    return f(x, wblob)


def kernel(input, E1, E2, E3, E4, E5, E6, E7, W):
    epk = jnp.concatenate([E1, E2, E3, E4, E5, E6, E7], axis=0)
    wrows = W.reshape(NT, D)[jnp.array(ROW_TABLE)]
    pad = ((0, RPAD - NROWS), (0, 0))
    et = jnp.pad(epk, pad).T.reshape(-1)
    wt = jnp.pad(wrows, pad).T.reshape(-1)
    wblob = jnp.concatenate([et, wt])
    out = _run(input, wblob)
    return out.reshape(B, 1)
